# Initial kernel scaffold; baseline (speedup 1.0000x reference)
import jax
import jax.numpy as jnp
from jax.experimental import pallas as pl


def kernel(x, edge_index, W, b):
    raise NotImplementedError("write your pallas kernel here")


# speedup vs baseline: 15.4054x; 15.4054x over previous
import functools

import jax
import jax.numpy as jnp
from jax import lax
from jax.experimental import pallas as pl
from jax.experimental.pallas import tpu as pltpu
from jax.experimental.pallas import tpu_sc as plsc

N = 10000
E = 320000
D = 128
NC = 2
NS = 16
NW = NC * NS
NP = 10240
ROWS = NP // D
EPW = E // NW
K = 128
EP = 327680
C = EP // (NS * K)
NBUF = 5
RPT = NP // NS

_mesh = plsc.VectorSubcoreMesh(core_axis_name="c", subcore_axis_name="s")


@functools.partial(
    pl.kernel,
    out_type=jax.ShapeDtypeStruct((NW, NP), jnp.float32),
    mesh=_mesh,
    scratch_types=[
        pltpu.VMEM((EPW,), jnp.int32),
        pltpu.VMEM((NP,), jnp.float32),
    ],
    compiler_params=pltpu.CompilerParams(needs_layout_passes=False),
)
def _sc_count(dst_hbm, z_hbm, cnt_hbm, dstv, cntv):
    c = lax.axis_index("c")
    s = lax.axis_index("s")
    w = c * NS + s
    pltpu.sync_copy(dst_hbm.at[w], dstv)
    pltpu.sync_copy(z_hbm, cntv)

    ones = jnp.ones((16,), jnp.float32)

    def body(j, carry):
        v = dstv[pl.ds(j * 16, 16)]
        plsc.addupdate_scatter(cntv, [v], ones)
        return carry

    lax.fori_loop(0, EPW // 16, body, 0)
    pltpu.sync_copy(cntv, cnt_hbm.at[w])


D2 = D // NC


@functools.partial(
    pl.kernel,
    out_type=jax.ShapeDtypeStruct((NC, NP, D2), jnp.float32),
    mesh=_mesh,
    scratch_types=[
        pltpu.VMEM((C, K), jnp.int32),
        pltpu.VMEM((C, K), jnp.int32),
        [pltpu.VMEM((K, D2), jnp.float32)] * NBUF,
        [pltpu.SemaphoreType.DMA] * NBUF,
        pltpu.VMEM_SHARED((NP, D2), jnp.float32),
    ],
    compiler_params=pltpu.CompilerParams(use_tc_tiling_on_sc=False),
)
def _sc_agg(g_hbm, src_hbm, dst_hbm, z_hbm, acc_hbm,
            srcv, dstv, rows, sems, acc_sh):
    c = lax.axis_index("c")
    s = lax.axis_index("s")
    w = c * NS + s
    pltpu.sync_copy(src_hbm.at[w], srcv)
    pltpu.sync_copy(dst_hbm.at[s], dstv)

    pltpu.sync_copy(z_hbm, rows[0])
    for k in range(RPT // K):
        pltpu.sync_copy(rows[0], acc_sh.at[pl.ds(s * RPT + k * K, K)])
    plsc.subcore_barrier()

    for b in range(NBUF):
        pltpu.async_copy(g_hbm.at[srcv.at[b]], rows[b], sems[b])

    def outer(o, carry):
        for b in range(NBUF):
            ch = o * NBUF + b
            pltpu.make_async_copy(g_hbm.at[srcv.at[ch]], rows[b],
                                  sems[b]).wait()
            pltpu.sync_copy(rows[b], acc_sh.at[dstv.at[ch]], add=True)
            nxt = ch + NBUF

            @pl.when(nxt < C)
            def _():
                pltpu.async_copy(g_hbm.at[srcv.at[nxt]], rows[b], sems[b])
        return carry

    lax.fori_loop(0, C // NBUF, outer, 0)
    plsc.subcore_barrier()
    pltpu.sync_copy(acc_sh.at[pl.ds(s * RPT, RPT)],
                    acc_hbm.at[c, pl.ds(s * RPT, RPT)])


BN = NP // 16


def _scale_body(x_ref, cnt_ref, g_ref):
    dinv = lax.rsqrt(jnp.sum(cnt_ref[...], axis=0) + 1.0)
    gx = x_ref[...] * dinv
    g_ref[0] = gx[:, :D2]
    g_ref[1] = gx[:, D2:]


def _final_body(g_ref, acc_ref, cnt_ref, w_ref, b_ref, o_ref):
    dinv = lax.rsqrt(jnp.sum(cnt_ref[...], axis=0) + 1.0)
    a = jnp.concatenate([acc_ref[0], acc_ref[1]], axis=1)
    gg = jnp.concatenate([g_ref[0], g_ref[1]], axis=1)
    t = (gg + a) * dinv
    o_ref[...] = lax.dot_general(
        t, w_ref[...], (((1,), (1,)), ((), ())),
        preferred_element_type=jnp.float32,
        precision=jax.lax.Precision.HIGHEST) + b_ref[...]


def kernel(x, edge_index, W, b):
    pad = EP - E
    srcp = jnp.concatenate([edge_index[0], jnp.zeros((pad,), jnp.int32)])
    dstp = jnp.concatenate([edge_index[1], jnp.full((pad,), N, jnp.int32)])
    base = srcp.reshape(NS, C, K)
    src3 = jnp.concatenate([base, base + NP]).reshape(NW, C, K)
    dst3 = dstp.reshape(NS, C, K)
    dst2 = edge_index[1].reshape(NW, EPW)
    zeros = jnp.zeros((K, D2), jnp.float32)
    zflat = jnp.zeros((NP,), jnp.float32)
    xp = jnp.pad(x, ((0, NP - N), (0, 0)))

    cnt = _sc_count(dst2, zflat).reshape(NW, NP, 1)

    g = pl.pallas_call(
        _scale_body,
        grid=(NP // BN,),
        in_specs=[
            pl.BlockSpec((BN, D), lambda i: (i, 0)),
            pl.BlockSpec((NW, BN, 1), lambda i: (0, i, 0)),
        ],
        out_specs=pl.BlockSpec((NC, BN, D2), lambda i: (0, i, 0)),
        out_shape=jax.ShapeDtypeStruct((NC, NP, D2), jnp.float32),
    )(xp, cnt)

    acc = _sc_agg(g.reshape(NC * NP, D2), src3, dst3, zeros)

    out = pl.pallas_call(
        _final_body,
        grid=(NP // BN,),
        in_specs=[
            pl.BlockSpec((NC, BN, D2), lambda i: (0, i, 0)),
            pl.BlockSpec((NC, BN, D2), lambda i: (0, i, 0)),
            pl.BlockSpec((NW, BN, 1), lambda i: (0, i, 0)),
            pl.BlockSpec((D, D), lambda i: (0, 0)),
            pl.BlockSpec((1, D), lambda i: (0, 0)),
        ],
        out_specs=pl.BlockSpec((BN, D), lambda i: (i, 0)),
        out_shape=jax.ShapeDtypeStruct((NP, D), jnp.float32),
    )(g, acc, cnt, W, b.reshape(1, D))

    return out[:N]
